# baseline (device time: 1582952 ns/iter reference)
import jax
import jax.numpy as jnp
from jax import lax
from jax.experimental import pallas as pl
from jax.experimental.pallas import tpu as pltpu

CAP = 1280
BM = 256


def _exchange_x(collective_id, *arrays):
    n = len(arrays)

    def body(*refs):
        in_refs = refs[:n]
        out_refs = refs[n : 2 * n]
        send_sems, recv_sems = refs[2 * n], refs[2 * n + 1]
        my_x = lax.axis_index("x")
        my_y = lax.axis_index("y")
        peer = (1 - my_x, my_y)

        barrier = pltpu.get_barrier_semaphore()
        pl.semaphore_signal(
            barrier, inc=1, device_id=peer, device_id_type=pl.DeviceIdType.MESH
        )
        pl.semaphore_wait(barrier, 1)

        rdmas = []
        for i in range(n):
            rdma = pltpu.make_async_remote_copy(
                src_ref=in_refs[i],
                dst_ref=out_refs[i],
                send_sem=send_sems.at[i],
                recv_sem=recv_sems.at[i],
                device_id=peer,
                device_id_type=pl.DeviceIdType.MESH,
            )
            rdma.start()
            rdmas.append(rdma)
        for rdma in rdmas:
            rdma.wait()

    return pl.pallas_call(
        body,
        out_shape=[jax.ShapeDtypeStruct(a.shape, a.dtype) for a in arrays],
        in_specs=[pl.BlockSpec(memory_space=pl.ANY)] * n,
        out_specs=[pl.BlockSpec(memory_space=pl.ANY)] * n,
        scratch_shapes=[
            pltpu.SemaphoreType.DMA((n,)),
            pltpu.SemaphoreType.DMA((n,)),
        ],
        compiler_params=pltpu.CompilerParams(collective_id=collective_id),
    )(*arrays)


def _grouped_gemm_relu(x, w, blocks_per_e, bm=BM, bn=1024):
    M, K = x.shape
    E, _, N = w.shape

    def body(x_ref, w_ref, o_ref):
        o_ref[...] = jnp.maximum(
            jnp.dot(
                x_ref[...].astype(jnp.bfloat16),
                w_ref[0].astype(jnp.bfloat16),
                preferred_element_type=jnp.float32,
            ),
            0.0,
        )

    return pl.pallas_call(
        body,
        grid=(N // bn, M // bm),
        in_specs=[
            pl.BlockSpec((bm, K), lambda n, m: (m, 0)),
            pl.BlockSpec((1, K, bn), lambda n, m: (m // blocks_per_e, 0, n)),
        ],
        out_specs=pl.BlockSpec((bm, bn), lambda n, m: (m, n)),
        out_shape=jax.ShapeDtypeStruct((M, N), jnp.float32),
    )(x, w)


def _grouped_gemm(x, w, blocks_per_e, bm=BM, bn=512):
    M, K = x.shape
    E, _, N = w.shape

    def body(x_ref, w_ref, o_ref):
        o_ref[...] = jnp.dot(
            x_ref[...].astype(jnp.bfloat16),
            w_ref[0].astype(jnp.bfloat16),
            preferred_element_type=jnp.float32,
        )

    return pl.pallas_call(
        body,
        grid=(N // bn, M // bm),
        in_specs=[
            pl.BlockSpec((bm, K), lambda n, m: (m, 0)),
            pl.BlockSpec((1, K, bn), lambda n, m: (m // blocks_per_e, 0, n)),
        ],
        out_specs=pl.BlockSpec((bm, bn), lambda n, m: (m, n)),
        out_shape=jax.ShapeDtypeStruct((M, N), jnp.float32),
    )(x, w)


def kernel(x, assign, W1, W2):
    T, D = x.shape
    E, _, F = W1.shape
    my_x = lax.axis_index("x")
    M = 2 * T
    S = E * CAP

    a2 = assign.reshape(T, 1)
    x_peer, a_peer = _exchange_x(0, x, a2)

    a_all = jnp.concatenate([a2, a_peer], axis=0)[:, 0]
    key = jnp.bitwise_xor(a_all, 4 * my_x)
    local = key < E

    onehot = (key[:, None] == jnp.arange(E)[None, :])
    ranks = jnp.cumsum(onehot.astype(jnp.int32), axis=0) - 1
    rank = jnp.sum(ranks * onehot, axis=1)
    valid = local & (rank < CAP)
    slot = jnp.where(valid, key * CAP + rank, S)

    x_sorted = jnp.zeros((S + 8, D), jnp.float32)
    x_sorted = x_sorted.at[slot[:T]].set(x)
    x_sorted = x_sorted.at[slot[T:]].set(x_peer)
    x_sorted = x_sorted[:S]

    h = _grouped_gemm_relu(x_sorted, W1, CAP // BM)
    y = _grouped_gemm(h, W2, CAP // BM)

    token_of_slot = jnp.full((S + 8,), M, jnp.int32)
    token_of_slot = token_of_slot.at[slot].set(jnp.arange(M, dtype=jnp.int32))
    token_of_slot = token_of_slot[:S]
    acc = jnp.zeros((M + 8, D), jnp.float32)
    acc = acc.at[token_of_slot].set(y)
    acc = acc[:M]

    (recv,) = _exchange_x(1, acc[T:])
    return acc[:T] + recv


# device time: 1359147 ns/iter; 1.1647x vs baseline; 1.1647x over previous
import jax
import jax.numpy as jnp
from jax import lax
from jax.experimental import pallas as pl
from jax.experimental.pallas import tpu as pltpu

CAP = 1280
CAPX = 2304
BM = 256


def _exchange_x(collective_id, *arrays):
    n = len(arrays)

    def body(*refs):
        in_refs = refs[:n]
        out_refs = refs[n : 2 * n]
        send_sems, recv_sems = refs[2 * n], refs[2 * n + 1]
        my_x = lax.axis_index("x")
        my_y = lax.axis_index("y")
        peer = (1 - my_x, my_y)

        barrier = pltpu.get_barrier_semaphore()
        pl.semaphore_signal(
            barrier, inc=1, device_id=peer, device_id_type=pl.DeviceIdType.MESH
        )
        pl.semaphore_wait(barrier, 1)

        rdmas = []
        for i in range(n):
            rdma = pltpu.make_async_remote_copy(
                src_ref=in_refs[i],
                dst_ref=out_refs[i],
                send_sem=send_sems.at[i],
                recv_sem=recv_sems.at[i],
                device_id=peer,
                device_id_type=pl.DeviceIdType.MESH,
            )
            rdma.start()
            rdmas.append(rdma)
        for rdma in rdmas:
            rdma.wait()

    return pl.pallas_call(
        body,
        out_shape=[jax.ShapeDtypeStruct(a.shape, a.dtype) for a in arrays],
        in_specs=[pl.BlockSpec(memory_space=pl.ANY)] * n,
        out_specs=[pl.BlockSpec(memory_space=pl.ANY)] * n,
        scratch_shapes=[
            pltpu.SemaphoreType.DMA((n,)),
            pltpu.SemaphoreType.DMA((n,)),
        ],
        compiler_params=pltpu.CompilerParams(collective_id=collective_id),
    )(*arrays)


def _grouped_gemm_relu(x, w, blocks_per_e, bm=BM, bn=1024):
    M, K = x.shape
    E, _, N = w.shape

    def body(x_ref, w_ref, o_ref):
        o_ref[...] = jnp.maximum(
            jnp.dot(x_ref[...], w_ref[0], preferred_element_type=jnp.float32),
            0.0,
        )

    return pl.pallas_call(
        body,
        grid=(N // bn, M // bm),
        in_specs=[
            pl.BlockSpec((bm, K), lambda n, m: (m, 0)),
            pl.BlockSpec((1, K, bn), lambda n, m: (m // blocks_per_e, 0, n)),
        ],
        out_specs=pl.BlockSpec((bm, bn), lambda n, m: (m, n)),
        out_shape=jax.ShapeDtypeStruct((M, N), jnp.float32),
    )(x, w)


def _grouped_gemm(x, w, blocks_per_e, bm=BM, bn=512):
    M, K = x.shape
    E, _, N = w.shape

    def body(x_ref, w_ref, o_ref):
        o_ref[...] = jnp.dot(
            x_ref[...], w_ref[0], preferred_element_type=jnp.float32
        )

    return pl.pallas_call(
        body,
        grid=(N // bn, M // bm),
        in_specs=[
            pl.BlockSpec((bm, K), lambda n, m: (m, 0)),
            pl.BlockSpec((1, K, bn), lambda n, m: (m // blocks_per_e, 0, n)),
        ],
        out_specs=pl.BlockSpec((bm, bn), lambda n, m: (m, n)),
        out_shape=jax.ShapeDtypeStruct((M, N), jnp.float32),
    )(x, w)


def _cap_rank(mask, cap):
    rank = jnp.cumsum(mask.astype(jnp.int32)) - 1
    return jnp.where(mask & (rank < cap), rank, cap)


def kernel(x, assign, W1, W2):
    T, D = x.shape
    E, _, F = W1.shape
    my_x = lax.axis_index("x")
    S = E * CAP
    NS = T + CAPX

    rel = assign - 4 * my_x
    is_local = (rel >= 0) & (rel < E)

    m_slot = _cap_rank(~is_local, CAPX)
    x_send = jnp.zeros((CAPX + 8, D), jnp.float32).at[m_slot].set(x)[:CAPX]
    a_send = (
        jnp.full((CAPX + 8, 1), -1, jnp.int32)
        .at[m_slot]
        .set(assign.reshape(T, 1))[:CAPX]
    )
    x_recv, a_recv = _exchange_x(0, x_send, a_send)

    key_own = jnp.where(is_local, rel, -1)
    ar = a_recv[:, 0]
    key_recv = jnp.where(ar >= 0, ar - 4 * my_x, -1)
    keys = jnp.concatenate([key_own, key_recv])

    onehot = keys[:, None] == jnp.arange(E)[None, :]
    ranks = jnp.cumsum(onehot.astype(jnp.int32), axis=0) - 1
    rank = jnp.sum(ranks * jnp.where(onehot, 1, 0), axis=1)
    validk = (keys >= 0) & (rank < CAP)
    slot = jnp.where(validk, keys * CAP + rank, S)

    x_sorted = (
        jnp.zeros((S + 8, D), jnp.float32)
        .at[slot[:T]]
        .set(x)
        .at[slot[T:]]
        .set(x_recv)[:S]
    )

    h = _grouped_gemm_relu(x_sorted, W1, CAP // BM)
    y = _grouped_gemm(h, W2, CAP // BM)

    tos = (
        jnp.full((S + 8,), NS, jnp.int32)
        .at[slot]
        .set(jnp.arange(NS, dtype=jnp.int32))[:S]
    )
    idx_own = jnp.where(tos < T, tos, T)
    out_buf = jnp.zeros((T + 8, D), jnp.float32).at[idx_own].set(y)

    idx_back = jnp.where(tos >= T, tos - T, CAPX)
    y_back = jnp.zeros((CAPX + 8, D), jnp.float32).at[idx_back].set(y)[:CAPX]

    (r_back,) = _exchange_x(1, y_back)

    tosend = (
        jnp.full((CAPX + 8,), T, jnp.int32)
        .at[m_slot]
        .set(jnp.arange(T, dtype=jnp.int32))[:CAPX]
    )
    out_buf = out_buf.at[tosend].set(r_back)
    return out_buf[:T]
